# baseline (speedup 1.0000x reference)
import jax
import jax.numpy as jnp
from jax import lax
from jax.experimental import pallas as pl
from jax.experimental.pallas import tpu as pltpu
from jax.experimental.pallas import tpu_sc as plsc

B, T, D = 16, 4096, 1024


def _body(x_hbm, lens_hbm, out_hbm, lens_v, idx_v, rows_v, sem):
    wid = lax.axis_index("s") * 2 + lax.axis_index("c")

    @pl.when(wid == 0)
    def _():
        pltpu.sync_copy(lens_hbm, lens_v)
        idx_v[...] = lax.iota(jnp.int32, B) * T + lens_v[...] - 1
        pltpu.async_copy(x_hbm.at[idx_v], rows_v, sem).wait()
        pltpu.sync_copy(rows_v, out_hbm)


_gather = pl.kernel(
    _body,
    out_type=jax.ShapeDtypeStruct((B, D), jnp.float32),
    mesh=plsc.VectorSubcoreMesh(core_axis_name="c", subcore_axis_name="s"),
    scratch_types=[
        pltpu.VMEM((B,), jnp.int32),
        pltpu.VMEM((B,), jnp.int32),
        pltpu.VMEM((B, D), jnp.float32),
        pltpu.SemaphoreType.DMA,
    ],
)


def kernel(x_unpacked, x_lens):
    x_flat = x_unpacked.reshape(B * T, D)
    lens32 = x_lens.astype(jnp.int32)
    return _gather(x_flat, lens32)


# speedup vs baseline: 1.1203x; 1.1203x over previous
import jax
import jax.numpy as jnp
from jax import lax
from jax.experimental import pallas as pl
from jax.experimental.pallas import tpu as pltpu
from jax.experimental.pallas import tpu_sc as plsc

B, T, D = 16, 4096, 1024


def _body(x_hbm, lens_hbm, out_hbm, lens_s, sem):
    pltpu.sync_copy(lens_hbm, lens_s)
    copies = []
    for b in range(B):
        t = lens_s[b] - 1
        copies.append(
            pltpu.make_async_copy(
                x_hbm.at[b, pl.ds(t, 1)], out_hbm.at[pl.ds(b, 1)], sem
            )
        )
    for c in copies:
        c.start()
    for c in copies:
        c.wait()


_gather = pl.kernel(
    _body,
    out_type=jax.ShapeDtypeStruct((B, D), jnp.float32),
    mesh=plsc.ScalarSubcoreMesh(axis_name="c", num_cores=1),
    scratch_types=[
        pltpu.SMEM((B,), jnp.int32),
        pltpu.SemaphoreType.DMA,
    ],
)


def kernel(x_unpacked, x_lens):
    lens32 = x_lens.astype(jnp.int32)
    return _gather(x_unpacked, lens32)
